# baseline (speedup 1.0000x reference)
import functools

import jax
import jax.numpy as jnp
from jax import lax
from jax.experimental import pallas as pl
from jax.experimental.pallas import tpu as pltpu
from jax.experimental.pallas import tpu_sc as plsc

_NC, _NS = 2, 16
_CHUNK = 125


def _sc_aggregate(h, src2d, dst2d, z):
    n, d = h.shape
    n_pad = ((n + 128 * _NS - 1) // (128 * _NS)) * (128 * _NS)
    rows_total = src2d.shape[0]
    rows_per_tile = rows_total // (_NC * _NS)
    stripe = n_pad // _NS
    assert z.shape[0] == stripe

    mesh = plsc.VectorSubcoreMesh(core_axis_name="c", subcore_axis_name="s")

    @functools.partial(
        pl.kernel,
        out_type=jax.ShapeDtypeStruct((_NC, n_pad, d), jnp.float32),
        mesh=mesh,
        scratch_types=[
            pltpu.VMEM((rows_per_tile, _CHUNK), jnp.int32),
            pltpu.VMEM((rows_per_tile, _CHUNK), jnp.int32),
            pltpu.VMEM((_CHUNK, d), jnp.float32),
            pltpu.VMEM_SHARED((n_pad, d), jnp.float32),
            pltpu.SemaphoreType.DMA,
        ],
    )
    def agg(h_hbm, src_hbm, dst_hbm, z_hbm, out_hbm,
            src_v, dst_v, rows_v, acc_sh, sem0):
        cid = lax.axis_index("c")
        sid = lax.axis_index("s")
        base = sid * stripe
        pltpu.sync_copy(z_hbm, acc_sh.at[pl.ds(base, stripe)])
        row0 = (cid * _NS + sid) * rows_per_tile
        pltpu.sync_copy(src_hbm.at[pl.ds(row0, rows_per_tile)], src_v)
        pltpu.sync_copy(dst_hbm.at[pl.ds(row0, rows_per_tile)], dst_v)
        plsc.subcore_barrier()

        @pl.loop(0, rows_per_tile)
        def _(j):
            pltpu.async_copy(h_hbm.at[src_v.at[j]], rows_v, sem0).wait()
            pltpu.sync_copy(rows_v, acc_sh.at[dst_v.at[j]], add=True)

        plsc.subcore_barrier()
        sl = pl.ds(base, stripe)
        pltpu.sync_copy(acc_sh.at[sl], out_hbm.at[cid].at[sl])

    return agg(h, src2d, dst2d, z)


def _fused_body(a_ref, nu_ref, w1_ref, wt_ref, b_ref, out_ref, feat_ref):
    h2u = jnp.dot(a_ref[...], w1_ref[...], preferred_element_type=jnp.float32)
    h2u = h2u * nu_ref[...]
    s = jnp.sum(h2u * h2u, axis=1, keepdims=True)
    inv = jax.lax.rsqrt(jnp.maximum(s, 1e-24))
    feat = h2u * inv
    feat_ref[...] = feat
    out_ref[...] = (
        jnp.dot(feat, wt_ref[...], preferred_element_type=jnp.float32) + b_ref[...]
    )


def kernel(x, edge_index, W1, W_out, b_out):
    n, d = x.shape
    c = W_out.shape[0]
    src = edge_index[0]
    dst = edge_index[1]

    cnt = jnp.zeros((n,), jnp.int32).at[dst].add(1)
    norm = jax.lax.rsqrt((cnt + 1).astype(x.dtype))

    h = x * norm[:, None]
    e = src.shape[0]
    n_pad = ((n + 128 * _NS - 1) // (128 * _NS)) * (128 * _NS)
    parts = _sc_aggregate(
        h,
        src.reshape(e // _CHUNK, _CHUNK),
        dst.reshape(e // _CHUNK, _CHUNK),
        jnp.zeros((n_pad // _NS, d), x.dtype),
    )

    present = (cnt > 0).astype(jnp.int32)
    ranks = jnp.cumsum(present) - 1
    u = (
        jnp.zeros((n,), dst.dtype)
        .at[jnp.where(present > 0, ranks, n)]
        .set(jnp.arange(n, dtype=dst.dtype), mode="drop")
    )

    a_u = jnp.take(parts[0, :n] + parts[1, :n] + h, u, axis=0)
    nu = jnp.take(norm, u)

    bm = 1000
    grid = (n // bm,)
    out, feat = pl.pallas_call(
        _fused_body,
        grid=grid,
        in_specs=[
            pl.BlockSpec((bm, d), lambda i: (i, 0)),
            pl.BlockSpec((bm, 1), lambda i: (i, 0)),
            pl.BlockSpec((d, d), lambda i: (0, 0)),
            pl.BlockSpec((d, c), lambda i: (0, 0)),
            pl.BlockSpec((1, c), lambda i: (0, 0)),
        ],
        out_specs=[
            pl.BlockSpec((bm, c), lambda i: (i, 0)),
            pl.BlockSpec((bm, d), lambda i: (i, 0)),
        ],
        out_shape=[
            jax.ShapeDtypeStruct((n, c), x.dtype),
            jax.ShapeDtypeStruct((n, d), x.dtype),
        ],
    )(a_u, nu[:, None], W1, W_out.T, b_out[None, :])
    return (out, feat)


# speedup vs baseline: 1.0147x; 1.0147x over previous
import functools

import jax
import jax.numpy as jnp
from jax import lax
from jax.experimental import pallas as pl
from jax.experimental.pallas import tpu as pltpu
from jax.experimental.pallas import tpu_sc as plsc

_NC, _NS = 2, 16
_CHUNK = 125


def _sc_aggregate(h, src2d, dst2d, z):
    n, d = h.shape
    n_pad = ((n + 128 * _NS - 1) // (128 * _NS)) * (128 * _NS)
    rows_total = src2d.shape[0]
    rows_per_tile = rows_total // (_NC * _NS)
    stripe = n_pad // _NS
    assert z.shape[0] == stripe

    mesh = plsc.VectorSubcoreMesh(core_axis_name="c", subcore_axis_name="s")

    @functools.partial(
        pl.kernel,
        out_type=jax.ShapeDtypeStruct((_NC, n_pad, d), jnp.float32),
        mesh=mesh,
        scratch_types=[
            pltpu.VMEM((rows_per_tile, _CHUNK), jnp.int32),
            pltpu.VMEM((rows_per_tile, _CHUNK), jnp.int32),
            pltpu.VMEM((_CHUNK, d), jnp.float32),
            pltpu.VMEM_SHARED((n_pad, d), jnp.float32),
            pltpu.SemaphoreType.DMA,
        ],
    )
    def agg(h_hbm, src_hbm, dst_hbm, z_hbm, out_hbm,
            src_v, dst_v, rows_v, acc_sh, sem0):
        cid = lax.axis_index("c")
        sid = lax.axis_index("s")
        base = sid * stripe
        pltpu.sync_copy(z_hbm, acc_sh.at[pl.ds(base, stripe)])
        row0 = (cid * _NS + sid) * rows_per_tile
        pltpu.sync_copy(src_hbm.at[pl.ds(row0, rows_per_tile)], src_v)
        pltpu.sync_copy(dst_hbm.at[pl.ds(row0, rows_per_tile)], dst_v)
        plsc.subcore_barrier()

        @pl.loop(0, rows_per_tile)
        def _(j):
            pltpu.async_copy(h_hbm.at[src_v.at[j]], rows_v, sem0).wait()
            pltpu.sync_copy(rows_v, acc_sh.at[dst_v.at[j]], add=True)

        plsc.subcore_barrier()
        sl = pl.ds(base, stripe)
        pltpu.sync_copy(acc_sh.at[sl], out_hbm.at[cid].at[sl])

    return agg(h, src2d, dst2d, z)


def _sg_mm_body(parts_ref, h_ref, norm_ref, w_ref, out_ref):
    a = parts_ref[0] + parts_ref[1] + h_ref[...]
    prod = jnp.dot(a, w_ref[...], preferred_element_type=jnp.float32)
    out_ref[...] = prod * norm_ref[...]


def _head_body(x2_ref, wt_ref, b_ref, out_ref, feat_ref):
    h2u = x2_ref[...]
    s = jnp.sum(h2u * h2u, axis=1, keepdims=True)
    inv = jax.lax.rsqrt(jnp.maximum(s, 1e-24))
    feat = h2u * inv
    feat_ref[...] = feat
    out_ref[...] = (
        jnp.dot(feat, wt_ref[...], preferred_element_type=jnp.float32) + b_ref[...]
    )


def kernel(x, edge_index, W1, W_out, b_out):
    n, d = x.shape
    c = W_out.shape[0]
    src = edge_index[0]
    dst = edge_index[1]

    cnt = jnp.zeros((n,), jnp.int32).at[dst].add(1)
    norm = jax.lax.rsqrt((cnt + 1).astype(x.dtype))

    h = x * norm[:, None]
    e = src.shape[0]
    n_pad = ((n + 128 * _NS - 1) // (128 * _NS)) * (128 * _NS)
    parts = _sc_aggregate(
        h,
        src.reshape(e // _CHUNK, _CHUNK),
        dst.reshape(e // _CHUNK, _CHUNK),
        jnp.zeros((n_pad // _NS, d), x.dtype),
    )

    bm = 1000
    grid = (n // bm,)
    h2 = pl.pallas_call(
        _sg_mm_body,
        grid=grid,
        in_specs=[
            pl.BlockSpec((2, bm, d), lambda i: (0, i, 0)),
            pl.BlockSpec((bm, d), lambda i: (i, 0)),
            pl.BlockSpec((bm, 1), lambda i: (i, 0)),
            pl.BlockSpec((d, d), lambda i: (0, 0)),
        ],
        out_specs=pl.BlockSpec((bm, d), lambda i: (i, 0)),
        out_shape=jax.ShapeDtypeStruct((n, d), x.dtype),
    )(parts, h, norm[:, None], W1)

    present = (cnt > 0).astype(jnp.int32)
    ranks = jnp.cumsum(present) - 1
    u = (
        jnp.zeros((n,), dst.dtype)
        .at[jnp.where(present > 0, ranks, n)]
        .set(jnp.arange(n, dtype=dst.dtype), mode="drop")
    )

    x2 = jnp.take(h2, u, axis=0)

    out, feat = pl.pallas_call(
        _head_body,
        grid=grid,
        in_specs=[
            pl.BlockSpec((bm, d), lambda i: (i, 0)),
            pl.BlockSpec((d, c), lambda i: (0, 0)),
            pl.BlockSpec((1, c), lambda i: (0, 0)),
        ],
        out_specs=[
            pl.BlockSpec((bm, c), lambda i: (i, 0)),
            pl.BlockSpec((bm, d), lambda i: (i, 0)),
        ],
        out_shape=[
            jax.ShapeDtypeStruct((n, c), x.dtype),
            jax.ShapeDtypeStruct((n, d), x.dtype),
        ],
    )(x2, W_out.T, b_out[None, :])
    return (out, feat)


# speedup vs baseline: 1.1352x; 1.1188x over previous
import functools

import jax
import jax.numpy as jnp
from jax import lax
from jax.experimental import pallas as pl
from jax.experimental.pallas import tpu as pltpu
from jax.experimental.pallas import tpu_sc as plsc

_NC, _NS = 2, 16
_CHUNK = 125


def _sc_histogram(dst, n_pad):
    e = dst.shape[0]
    ept = e // (_NC * _NS)
    stripe = n_pad // _NS
    L = 16
    mesh = plsc.VectorSubcoreMesh(core_axis_name="c", subcore_axis_name="s")

    @functools.partial(
        pl.kernel,
        out_type=jax.ShapeDtypeStruct((2 * n_pad,), jnp.float32),
        mesh=mesh,
        compiler_params=pltpu.CompilerParams(needs_layout_passes=False),
        scratch_types=[
            pltpu.VMEM((ept,), jnp.int32),
            pltpu.VMEM((n_pad,), jnp.float32),
            pltpu.VMEM((_NS, stripe), jnp.float32),
            pltpu.VMEM((stripe,), jnp.float32),
            pltpu.VMEM_SHARED((_NS, n_pad), jnp.float32),
            pltpu.SemaphoreType.DMA,
        ],
    )
    def hist(dst_hbm, out_hbm, dst_v, hist_v, slab_v, tot_v, sh, sem):
        cid = lax.axis_index("c")
        sid = lax.axis_index("s")
        pltpu.sync_copy(dst_hbm.at[pl.ds((cid * _NS + sid) * ept, ept)], dst_v)

        zeros = jnp.zeros((L,), jnp.float32)

        @pl.loop(0, n_pad // L)
        def _(i):
            hist_v[pl.ds(i * L, L)] = zeros

        ones = jnp.ones((L,), jnp.float32)

        @pl.loop(0, ept // L)
        def _(k):
            idx = dst_v[pl.ds(k * L, L)]
            plsc.addupdate_scatter(hist_v, [idx], ones)

        pltpu.sync_copy(hist_v, sh.at[sid])
        plsc.subcore_barrier()
        pltpu.sync_copy(sh.at[:, pl.ds(sid * stripe, stripe)], slab_v)

        @pl.loop(0, stripe // L)
        def _(r):
            sl = pl.ds(r * L, L)
            acc = slab_v[0, sl]
            for t in range(1, _NS):
                acc = acc + slab_v[t, sl]
            tot_v[sl] = acc

        pltpu.sync_copy(
            tot_v, out_hbm.at[pl.ds(cid * n_pad + sid * stripe, stripe)])

    return hist(dst)


def _sc_aggregate(h, src2d, dst2d, z):
    n, d = h.shape
    n_pad = ((n + 128 * _NS - 1) // (128 * _NS)) * (128 * _NS)
    rows_total = src2d.shape[0]
    rows_per_tile = rows_total // (_NC * _NS)
    stripe = n_pad // _NS
    assert z.shape[0] == stripe

    mesh = plsc.VectorSubcoreMesh(core_axis_name="c", subcore_axis_name="s")

    @functools.partial(
        pl.kernel,
        out_type=jax.ShapeDtypeStruct((_NC, n_pad, d), jnp.float32),
        mesh=mesh,
        scratch_types=[
            pltpu.VMEM((rows_per_tile, _CHUNK), jnp.int32),
            pltpu.VMEM((rows_per_tile, _CHUNK), jnp.int32),
            pltpu.VMEM((_CHUNK, d), jnp.float32),
            pltpu.VMEM_SHARED((n_pad, d), jnp.float32),
            pltpu.SemaphoreType.DMA,
        ],
    )
    def agg(h_hbm, src_hbm, dst_hbm, z_hbm, out_hbm,
            src_v, dst_v, rows_v, acc_sh, sem0):
        cid = lax.axis_index("c")
        sid = lax.axis_index("s")
        base = sid * stripe
        pltpu.sync_copy(z_hbm, acc_sh.at[pl.ds(base, stripe)])
        row0 = (cid * _NS + sid) * rows_per_tile
        pltpu.sync_copy(src_hbm.at[pl.ds(row0, rows_per_tile)], src_v)
        pltpu.sync_copy(dst_hbm.at[pl.ds(row0, rows_per_tile)], dst_v)
        plsc.subcore_barrier()

        @pl.loop(0, rows_per_tile)
        def _(j):
            pltpu.async_copy(h_hbm.at[src_v.at[j]], rows_v, sem0).wait()
            pltpu.sync_copy(rows_v, acc_sh.at[dst_v.at[j]], add=True)

        plsc.subcore_barrier()
        sl = pl.ds(base, stripe)
        pltpu.sync_copy(acc_sh.at[sl], out_hbm.at[cid].at[sl])

    return agg(h, src2d, dst2d, z)


def _sg_mm_body(parts_ref, h_ref, norm_ref, w_ref, out_ref):
    a = parts_ref[0] + parts_ref[1] + h_ref[...]
    prod = jnp.dot(a, w_ref[...], preferred_element_type=jnp.float32)
    out_ref[...] = prod * norm_ref[...]


def _head_body(x2_ref, wt_ref, b_ref, out_ref, feat_ref):
    h2u = x2_ref[...]
    s = jnp.sum(h2u * h2u, axis=1, keepdims=True)
    inv = jax.lax.rsqrt(jnp.maximum(s, 1e-24))
    feat = h2u * inv
    feat_ref[...] = feat
    out_ref[...] = (
        jnp.dot(feat, wt_ref[...], preferred_element_type=jnp.float32) + b_ref[...]
    )


def kernel(x, edge_index, W1, W_out, b_out):
    n, d = x.shape
    c = W_out.shape[0]
    src = edge_index[0]
    dst = edge_index[1]

    n_pad = ((n + 128 * _NS - 1) // (128 * _NS)) * (128 * _NS)
    hist2 = _sc_histogram(dst, n_pad)
    cntf = hist2[:n] + hist2[n_pad:n_pad + n]
    norm = jax.lax.rsqrt(cntf + 1.0)

    h = x * norm[:, None]
    e = src.shape[0]
    parts = _sc_aggregate(
        h,
        src.reshape(e // _CHUNK, _CHUNK),
        dst.reshape(e // _CHUNK, _CHUNK),
        jnp.zeros((n_pad // _NS, d), x.dtype),
    )

    bm = 1000
    grid = (n // bm,)
    h2 = pl.pallas_call(
        _sg_mm_body,
        grid=grid,
        in_specs=[
            pl.BlockSpec((2, bm, d), lambda i: (0, i, 0)),
            pl.BlockSpec((bm, d), lambda i: (i, 0)),
            pl.BlockSpec((bm, 1), lambda i: (i, 0)),
            pl.BlockSpec((d, d), lambda i: (0, 0)),
        ],
        out_specs=pl.BlockSpec((bm, d), lambda i: (i, 0)),
        out_shape=jax.ShapeDtypeStruct((n, d), x.dtype),
    )(parts, h, norm[:, None], W1)

    present = (cntf > 0).astype(jnp.int32)
    ranks = jnp.cumsum(present) - 1
    u = (
        jnp.zeros((n,), dst.dtype)
        .at[jnp.where(present > 0, ranks, n)]
        .set(jnp.arange(n, dtype=dst.dtype), mode="drop")
    )

    x2 = h2.at[u].get(mode="promise_in_bounds")

    out, feat = pl.pallas_call(
        _head_body,
        grid=grid,
        in_specs=[
            pl.BlockSpec((bm, d), lambda i: (i, 0)),
            pl.BlockSpec((d, c), lambda i: (0, 0)),
            pl.BlockSpec((1, c), lambda i: (0, 0)),
        ],
        out_specs=[
            pl.BlockSpec((bm, c), lambda i: (i, 0)),
            pl.BlockSpec((bm, d), lambda i: (i, 0)),
        ],
        out_shape=[
            jax.ShapeDtypeStruct((n, c), x.dtype),
            jax.ShapeDtypeStruct((n, d), x.dtype),
        ],
    )(x2, W_out.T, b_out[None, :])
    return (out, feat)


# speedup vs baseline: 1.1762x; 1.0361x over previous
import functools

import jax
import jax.numpy as jnp
from jax import lax
from jax.experimental import pallas as pl
from jax.experimental.pallas import tpu as pltpu
from jax.experimental.pallas import tpu_sc as plsc

_NC, _NS = 2, 16
_CHUNK = 125


def _sc_histogram(ei_flat, e, n_pad):
    ept = e // (_NC * _NS)
    stripe = n_pad // _NS
    L = 16
    mesh = plsc.VectorSubcoreMesh(core_axis_name="c", subcore_axis_name="s")

    @functools.partial(
        pl.kernel,
        out_type=jax.ShapeDtypeStruct((2 * n_pad,), jnp.float32),
        mesh=mesh,
        compiler_params=pltpu.CompilerParams(needs_layout_passes=False),
        scratch_types=[
            pltpu.VMEM((ept,), jnp.int32),
            pltpu.VMEM((n_pad,), jnp.float32),
            pltpu.VMEM((_NS, stripe), jnp.float32),
            pltpu.VMEM((stripe,), jnp.float32),
            pltpu.VMEM_SHARED((_NS, n_pad), jnp.float32),
            pltpu.SemaphoreType.DMA,
        ],
    )
    def hist(ei_hbm, out_hbm, dst_v, hist_v, slab_v, tot_v, sh, sem):
        cid = lax.axis_index("c")
        sid = lax.axis_index("s")
        pltpu.sync_copy(
            ei_hbm.at[pl.ds(e + (cid * _NS + sid) * ept, ept)], dst_v)

        zeros = jnp.zeros((L,), jnp.float32)

        @pl.loop(0, n_pad // L)
        def _(i):
            hist_v[pl.ds(i * L, L)] = zeros

        ones = jnp.ones((L,), jnp.float32)

        @pl.loop(0, ept // L)
        def _(k):
            idx = dst_v[pl.ds(k * L, L)]
            plsc.addupdate_scatter(hist_v, [idx], ones)

        pltpu.sync_copy(hist_v, sh.at[sid])
        plsc.subcore_barrier()
        pltpu.sync_copy(sh.at[:, pl.ds(sid * stripe, stripe)], slab_v)

        @pl.loop(0, stripe // L)
        def _(r):
            sl = pl.ds(r * L, L)
            acc = slab_v[0, sl]
            for t in range(1, _NS):
                acc = acc + slab_v[t, sl]
            tot_v[sl] = acc

        pltpu.sync_copy(
            tot_v, out_hbm.at[pl.ds(cid * n_pad + sid * stripe, stripe)])

    return hist(ei_flat)


def _sc_aggregate(h, ei3, z):
    n, d = h.shape
    n_pad = ((n + 128 * _NS - 1) // (128 * _NS)) * (128 * _NS)
    rows_total = ei3.shape[1]
    rows_per_tile = rows_total // (_NC * _NS)
    stripe = n_pad // _NS
    assert z.shape[0] == stripe

    mesh = plsc.VectorSubcoreMesh(core_axis_name="c", subcore_axis_name="s")

    @functools.partial(
        pl.kernel,
        out_type=jax.ShapeDtypeStruct((_NC, n_pad, d), jnp.float32),
        mesh=mesh,
        scratch_types=[
            pltpu.VMEM((rows_per_tile, _CHUNK), jnp.int32),
            pltpu.VMEM((rows_per_tile, _CHUNK), jnp.int32),
            pltpu.VMEM((_CHUNK, d), jnp.float32),
            pltpu.VMEM_SHARED((n_pad, d), jnp.float32),
            pltpu.SemaphoreType.DMA,
        ],
    )
    def agg(h_hbm, ei_hbm, z_hbm, out_hbm,
            src_v, dst_v, rows_v, acc_sh, sem0):
        cid = lax.axis_index("c")
        sid = lax.axis_index("s")
        base = sid * stripe
        pltpu.sync_copy(z_hbm, acc_sh.at[pl.ds(base, stripe)])
        row0 = (cid * _NS + sid) * rows_per_tile
        sl_rows = pl.ds(row0, rows_per_tile)
        pltpu.sync_copy(ei_hbm.at[0].at[sl_rows], src_v)
        pltpu.sync_copy(ei_hbm.at[1].at[sl_rows], dst_v)
        plsc.subcore_barrier()

        @pl.loop(0, rows_per_tile)
        def _(j):
            pltpu.async_copy(h_hbm.at[src_v.at[j]], rows_v, sem0).wait()
            pltpu.sync_copy(rows_v, acc_sh.at[dst_v.at[j]], add=True)

        plsc.subcore_barrier()
        sl = pl.ds(base, stripe)
        pltpu.sync_copy(acc_sh.at[sl], out_hbm.at[cid].at[sl])

    return agg(h, ei3, z)


def _sg_mm_body(parts_ref, h_ref, norm_ref, w_ref, out_ref):
    a = parts_ref[0] + parts_ref[1] + h_ref[...]
    prod = jnp.dot(a, w_ref[...], preferred_element_type=jnp.float32)
    out_ref[...] = prod * norm_ref[...]


def _head_body(x2_ref, wt_ref, b_ref, out_ref, feat_ref):
    h2u = x2_ref[...]
    s = jnp.sum(h2u * h2u, axis=1, keepdims=True)
    inv = jax.lax.rsqrt(jnp.maximum(s, 1e-24))
    feat = h2u * inv
    feat_ref[...] = feat
    out_ref[...] = (
        jnp.dot(feat, wt_ref[...], preferred_element_type=jnp.float32) + b_ref[...]
    )


def kernel(x, edge_index, W1, W_out, b_out):
    n, d = x.shape
    c = W_out.shape[0]
    e = edge_index.shape[1]

    n_pad = ((n + 128 * _NS - 1) // (128 * _NS)) * (128 * _NS)
    hist2 = _sc_histogram(edge_index.reshape(2 * e), e, n_pad)
    cntf = hist2[:n] + hist2[n_pad:n_pad + n]
    norm = jax.lax.rsqrt(cntf + 1.0)

    h = x * norm[:, None]
    parts = _sc_aggregate(
        h,
        edge_index.reshape(2, e // _CHUNK, _CHUNK),
        jnp.zeros((n_pad // _NS, d), x.dtype),
    )

    bm = 1000
    grid = (n // bm,)
    h2 = pl.pallas_call(
        _sg_mm_body,
        grid=grid,
        in_specs=[
            pl.BlockSpec((2, bm, d), lambda i: (0, i, 0)),
            pl.BlockSpec((bm, d), lambda i: (i, 0)),
            pl.BlockSpec((bm, 1), lambda i: (i, 0)),
            pl.BlockSpec((d, d), lambda i: (0, 0)),
        ],
        out_specs=pl.BlockSpec((bm, d), lambda i: (i, 0)),
        out_shape=jax.ShapeDtypeStruct((n, d), x.dtype),
    )(parts, h, norm[:, None], W1)

    present = (cntf > 0).astype(jnp.int32)
    ranks = jnp.cumsum(present) - 1
    u = (
        jnp.zeros((n,), edge_index.dtype)
        .at[jnp.where(present > 0, ranks, n)]
        .set(jnp.arange(n, dtype=edge_index.dtype), mode="drop")
    )

    x2 = h2.at[u].get(mode="promise_in_bounds")

    out, feat = pl.pallas_call(
        _head_body,
        grid=grid,
        in_specs=[
            pl.BlockSpec((bm, d), lambda i: (i, 0)),
            pl.BlockSpec((d, c), lambda i: (0, 0)),
            pl.BlockSpec((1, c), lambda i: (0, 0)),
        ],
        out_specs=[
            pl.BlockSpec((bm, c), lambda i: (i, 0)),
            pl.BlockSpec((bm, d), lambda i: (i, 0)),
        ],
        out_shape=[
            jax.ShapeDtypeStruct((n, c), x.dtype),
            jax.ShapeDtypeStruct((n, d), x.dtype),
        ],
    )(x2, W_out.T, b_out[None, :])
    return (out, feat)
